# baseline (speedup 1.0000x reference)
import functools

import jax
import jax.numpy as jnp
from jax.experimental import pallas as pl
from jax.experimental.pallas import tpu as pltpu
from jax.experimental.pallas import tpu_sc as plsc

N_E = 8192
E_DIM = 256
ST = 128
BETA = 0.25
ENT_RATIO = 0.1

_R = 128
_NROWS = 16384
_NB = _NROWS // _R
_SC_CHUNK = 128


def _l2n(x):
    return x / jnp.clip(jnp.linalg.norm(x, axis=-1, keepdims=True), 1e-12, None)


def _vq_body(znc_ref, ec_ref,
             stt_ref, stg_ref, sz_ref, set_ref, seg_ref, sec_ref,
             idx_ref, scal_ref,
             accp_s, sacc_s, gt_s, gg_s, esw_s, ssc_s):
    i = pl.program_id(0)

    @pl.when(i == 0)
    def _init():
        accp_s[...] = jnp.zeros_like(accp_s)
        sacc_s[0] = 0.0
        sacc_s[1] = 0.0
        sacc_s[2] = 0.0
        sacc_s[3] = 0.0
        et = ec_ref[:, :ST]
        eg = ec_ref[:, ST:]
        gt_s[...] = jax.lax.dot_general(et, et, (((0,), (0,)), ((), ())),
                                        preferred_element_type=jnp.float32)
        gg_s[...] = jax.lax.dot_general(eg, eg, (((0,), (0,)), ((), ())),
                                        preferred_element_type=jnp.float32)
        ones_r = jnp.ones((1, N_E), jnp.float32)
        esw_s[0:1, :] = jax.lax.dot_general(
            ones_r, et, (((1,), (0,)), ((), ())),
            preferred_element_type=jnp.float32)
        esw_s[1:2, :] = jax.lax.dot_general(
            set_ref[...], et, (((1,), (0,)), ((), ())),
            preferred_element_type=jnp.float32)
        esw_s[2:3, :] = jax.lax.dot_general(
            ones_r, eg, (((1,), (0,)), ((), ())),
            preferred_element_type=jnp.float32)
        esw_s[3:4, :] = jax.lax.dot_general(
            seg_ref[...], eg, (((1,), (0,)), ((), ())),
            preferred_element_type=jnp.float32)
        se_t = set_ref[...]
        se_g = seg_ref[...]
        ssc_s[0] = jnp.sum(se_t)
        ssc_s[1] = jnp.sum(se_t * se_t)
        ssc_s[2] = jnp.sum(se_g)
        ssc_s[3] = jnp.sum(se_g * se_g)

    znc = znc_ref[...]
    ztn = znc[:, :ST]
    zgn = znc[:, ST:]
    ip = jax.lax.dot_general(znc, ec_ref[...], (((1,), (1,)), ((), ())),
                             preferred_element_type=jnp.float32)
    d = (sz_ref[...] + sec_ref[...]) - 2.0 * ip

    stt_v = stt_ref[...]
    stg_v = stg_ref[...]
    ut = jax.lax.dot_general(ztn, esw_s[0:2, :], (((1,), (1,)), ((), ())),
                             preferred_element_type=jnp.float32)
    ug = jax.lax.dot_general(zgn, esw_s[2:4, :], (((1,), (1,)), ((), ())),
                             preferred_element_type=jnp.float32)
    ht = jax.lax.dot_general(ztn, gt_s[...], (((1,), (0,)), ((), ())),
                             preferred_element_type=jnp.float32)
    hg = jax.lax.dot_general(zgn, gg_s[...], (((1,), (0,)), ((), ())),
                             preferred_element_type=jnp.float32)
    cross_t = jnp.sum(stt_v * ut[:, 0:1]) + jnp.sum(ut[:, 1:2])
    cross_g = jnp.sum(stg_v * ug[:, 0:1]) + jnp.sum(ug[:, 1:2])
    gram_t = jnp.sum(ht * ztn)
    gram_g = jnp.sum(hg * zgn)
    pdt = (N_E * jnp.sum(stt_v * stt_v) + _R * ssc_s[1]
           + 2.0 * jnp.sum(stt_v) * ssc_s[0]) - 4.0 * cross_t + 4.0 * gram_t
    pdg = (N_E * jnp.sum(stg_v * stg_v) + _R * ssc_s[3]
           + 2.0 * jnp.sum(stg_v) * ssc_s[2]) - 4.0 * cross_g + 4.0 * gram_g

    minv = jnp.min(d, axis=1, keepdims=True)
    midx = jnp.argmin(d, axis=1).astype(jnp.int32)
    idx_ref[...] = midx.reshape(1, 1, _R)

    pvq = jnp.sum(minv)

    md = minv - d
    e = jnp.exp(md * 100.0)
    s = jnp.sum(e, axis=1, keepdims=True)
    t = jnp.sum(e * md, axis=1, keepdims=True)
    rinv = 1.0 / s
    plogp = t * rinv * 100.0 - jnp.log(s)
    psum = jnp.sum(plogp)
    accp_s[...] += jax.lax.dot_general(rinv, e, (((0,), (0,)), ((), ())),
                                       preferred_element_type=jnp.float32)

    sacc_s[0] += pdt
    sacc_s[1] += pdg
    sacc_s[2] += pvq
    sacc_s[3] += psum

    @pl.when(i == _NB - 1)
    def _fin():
        avg = accp_s[...] * (1.0 / _NROWS)
        avg_ent = -jnp.sum(avg * jnp.log(avg + 1e-05))
        sample_ent = -(sacc_s[3] / _NROWS)
        vq = sacc_s[2] / (_NROWS * 256.0)
        scal_ref[0] = vq
        scal_ref[1] = BETA * vq
        scal_ref[2] = ENT_RATIO * (sample_ent - avg_ent)
        scal_ref[3] = sacc_s[0] / _NROWS
        scal_ref[4] = sacc_s[1] / _NROWS


def _make_sc_gather():
    info = plsc.get_sparse_core_info()
    nw = info.num_cores * info.num_subcores
    bpw = _NROWS // nw
    nch = bpw // _SC_CHUNK
    mesh = plsc.VectorSubcoreMesh(core_axis_name="c", subcore_axis_name="s")

    @functools.partial(
        pl.kernel, mesh=mesh,
        out_type=jax.ShapeDtypeStruct((_NROWS, E_DIM), jnp.float32),
        scratch_types=[
            pltpu.VMEM((_SC_CHUNK,), jnp.int32),
            pltpu.VMEM((_SC_CHUNK, E_DIM), jnp.float32),
            pltpu.SemaphoreType.DMA,
        ],
    )
    def gather_k(table_hbm, idx_hbm, out_hbm, idx_v, rows_v, sem):
        wid = jax.lax.axis_index("s") * info.num_cores + jax.lax.axis_index("c")
        base = wid * bpw
        for c in range(nch):
            off = base + c * _SC_CHUNK
            pltpu.sync_copy(idx_hbm.at[pl.ds(off, _SC_CHUNK)], idx_v)
            pltpu.async_copy(table_hbm.at[idx_v], rows_v, sem).wait()
            pltpu.sync_copy(rows_v, out_hbm.at[pl.ds(off, _SC_CHUNK)])

    return gather_k


def kernel(z, emb_text, emb_graph):
    z_flat = z.reshape(-1, E_DIM)
    zf_text = z_flat[:, :ST]
    zf_graph = z_flat[:, ST:]
    zf_text_n = _l2n(zf_text)
    zf_graph_n = _l2n(zf_graph)
    et_n = _l2n(emb_text)
    eg_n = _l2n(emb_graph)

    stt = jnp.sum(zf_text_n ** 2, axis=1, keepdims=True)
    stg = jnp.sum(zf_graph_n ** 2, axis=1, keepdims=True)
    se_t = jnp.sum(et_n ** 2, axis=1).reshape(1, N_E)
    se_g = jnp.sum(eg_n ** 2, axis=1).reshape(1, N_E)
    sz = stt + stg
    sec = se_t + se_g
    znc = jnp.concatenate([zf_text_n, zf_graph_n], axis=-1)
    all_emb = jnp.concatenate([et_n, eg_n], axis=-1)

    idx3, scal = pl.pallas_call(
        _vq_body,
        grid=(_NB,),
        in_specs=[
            pl.BlockSpec((_R, E_DIM), lambda i: (i, 0)),
            pl.BlockSpec((N_E, E_DIM), lambda i: (0, 0)),
            pl.BlockSpec((_R, 1), lambda i: (i, 0)),
            pl.BlockSpec((_R, 1), lambda i: (i, 0)),
            pl.BlockSpec((_R, 1), lambda i: (i, 0)),
            pl.BlockSpec((1, N_E), lambda i: (0, 0)),
            pl.BlockSpec((1, N_E), lambda i: (0, 0)),
            pl.BlockSpec((1, N_E), lambda i: (0, 0)),
        ],
        out_specs=[
            pl.BlockSpec((1, 1, _R), lambda i: (i, 0, 0)),
            pl.BlockSpec(memory_space=pltpu.SMEM),
        ],
        out_shape=[
            jax.ShapeDtypeStruct((_NB, 1, _R), jnp.int32),
            jax.ShapeDtypeStruct((8,), jnp.float32),
        ],
        scratch_shapes=[
            pltpu.VMEM((1, N_E), jnp.float32),
            pltpu.SMEM((8,), jnp.float32),
            pltpu.VMEM((ST, ST), jnp.float32),
            pltpu.VMEM((ST, ST), jnp.float32),
            pltpu.VMEM((4, ST), jnp.float32),
            pltpu.SMEM((4,), jnp.float32),
        ],
    )(znc, all_emb, stt, stg, sz, se_t, se_g, sec)

    min_idx = idx3.reshape(_NROWS)
    zq = _make_sc_gather()(all_emb, min_idx)

    z_q_out = zq.reshape(z.shape)
    z_q_text = zq[:, :ST].reshape(z.shape[0], z.shape[1], ST)
    z_q_graph = zq[:, ST:].reshape(z.shape[0], z.shape[1], ST)
    return (z_q_out, scal[0], scal[1], scal[2], scal[3], scal[4],
            z_q_text, z_q_graph, min_idx)


# speedup vs baseline: 1.0184x; 1.0184x over previous
import functools

import jax
import jax.numpy as jnp
from jax.experimental import pallas as pl
from jax.experimental.pallas import tpu as pltpu
from jax.experimental.pallas import tpu_sc as plsc

N_E = 8192
E_DIM = 256
ST = 128
BETA = 0.25
ENT_RATIO = 0.1

_R = 128
_NROWS = 16384
_NB = _NROWS // _R
_SC_CHUNK = 128


def _l2n(x):
    return x / jnp.clip(jnp.linalg.norm(x, axis=-1, keepdims=True), 1e-12, None)


def _vq_body(znc_ref, ecm2_ref,
             stt_ref, stg_ref, sz_ref, set_ref, seg_ref, sec_ref,
             idx_ref, scal_ref,
             accp_s, sacc_s, gt_s, gg_s, esw_s, ssc_s, iota_s):
    i = pl.program_id(0)

    @pl.when(i == 0)
    def _init():
        accp_s[...] = jnp.zeros_like(accp_s)
        sacc_s[0] = 0.0
        sacc_s[1] = 0.0
        sacc_s[2] = 0.0
        sacc_s[3] = 0.0
        et = ecm2_ref[:, :ST]
        eg = ecm2_ref[:, ST:]
        iota_s[...] = jax.lax.broadcasted_iota(
            jnp.int32, (1, N_E), 1).astype(jnp.float32)
        gt_s[...] = jax.lax.dot_general(et, et, (((0,), (0,)), ((), ())),
                                        preferred_element_type=jnp.float32)
        gg_s[...] = jax.lax.dot_general(eg, eg, (((0,), (0,)), ((), ())),
                                        preferred_element_type=jnp.float32)
        ones_r = jnp.ones((1, N_E), jnp.float32)
        esw_s[0:1, :] = jax.lax.dot_general(
            ones_r, et, (((1,), (0,)), ((), ())),
            preferred_element_type=jnp.float32)
        esw_s[1:2, :] = jax.lax.dot_general(
            set_ref[...], et, (((1,), (0,)), ((), ())),
            preferred_element_type=jnp.float32)
        esw_s[2:3, :] = jax.lax.dot_general(
            ones_r, eg, (((1,), (0,)), ((), ())),
            preferred_element_type=jnp.float32)
        esw_s[3:4, :] = jax.lax.dot_general(
            seg_ref[...], eg, (((1,), (0,)), ((), ())),
            preferred_element_type=jnp.float32)
        se_t = set_ref[...]
        se_g = seg_ref[...]
        ssc_s[0] = jnp.sum(se_t)
        ssc_s[1] = jnp.sum(se_t * se_t)
        ssc_s[2] = jnp.sum(se_g)
        ssc_s[3] = jnp.sum(se_g * se_g)

    znc = znc_ref[...]
    ztn = znc[:, :ST]
    zgn = znc[:, ST:]
    ip = jax.lax.dot_general(znc, ecm2_ref[...], (((1,), (1,)), ((), ())),
                             preferred_element_type=jnp.float32)
    d = (sz_ref[...] + sec_ref[...]) + ip

    stt_v = stt_ref[...]
    stg_v = stg_ref[...]
    ut = jax.lax.dot_general(ztn, esw_s[0:2, :], (((1,), (1,)), ((), ())),
                             preferred_element_type=jnp.float32)
    ug = jax.lax.dot_general(zgn, esw_s[2:4, :], (((1,), (1,)), ((), ())),
                             preferred_element_type=jnp.float32)
    ht = jax.lax.dot_general(ztn, gt_s[...], (((1,), (0,)), ((), ())),
                             preferred_element_type=jnp.float32)
    hg = jax.lax.dot_general(zgn, gg_s[...], (((1,), (0,)), ((), ())),
                             preferred_element_type=jnp.float32)
    cross_t = jnp.sum(stt_v * ut[:, 0:1]) + jnp.sum(ut[:, 1:2])
    cross_g = jnp.sum(stg_v * ug[:, 0:1]) + jnp.sum(ug[:, 1:2])
    gram_t = jnp.sum(ht * ztn)
    gram_g = jnp.sum(hg * zgn)
    pdt = (N_E * jnp.sum(stt_v * stt_v) + _R * ssc_s[1]
           + 2.0 * jnp.sum(stt_v) * ssc_s[0]) + 2.0 * cross_t + gram_t
    pdg = (N_E * jnp.sum(stg_v * stg_v) + _R * ssc_s[3]
           + 2.0 * jnp.sum(stg_v) * ssc_s[2]) + 2.0 * cross_g + gram_g

    minv = jnp.min(d, axis=1, keepdims=True)
    cand = jnp.where(d == minv, iota_s[...], float(N_E))
    midx = jnp.min(cand, axis=1).astype(jnp.int32)
    idx_ref[...] = midx.reshape(1, 1, _R)

    pvq = jnp.sum(minv)

    md = minv - d
    e = jnp.exp(md * 100.0)
    s = jnp.sum(e, axis=1, keepdims=True)
    t = jnp.sum(e * md, axis=1, keepdims=True)
    rinv = 1.0 / s
    plogp = t * rinv * 100.0 - jnp.log(s)
    psum = jnp.sum(plogp)
    accp_s[...] += jax.lax.dot_general(rinv, e, (((0,), (0,)), ((), ())),
                                       preferred_element_type=jnp.float32)

    sacc_s[0] += pdt
    sacc_s[1] += pdg
    sacc_s[2] += pvq
    sacc_s[3] += psum

    @pl.when(i == _NB - 1)
    def _fin():
        avg = accp_s[...] * (1.0 / _NROWS)
        avg_ent = -jnp.sum(avg * jnp.log(avg + 1e-05))
        sample_ent = -(sacc_s[3] / _NROWS)
        vq = sacc_s[2] / (_NROWS * 256.0)
        scal_ref[0] = vq
        scal_ref[1] = BETA * vq
        scal_ref[2] = ENT_RATIO * (sample_ent - avg_ent)
        scal_ref[3] = sacc_s[0] / _NROWS
        scal_ref[4] = sacc_s[1] / _NROWS


def _make_sc_gather():
    info = plsc.get_sparse_core_info()
    nw = info.num_cores * info.num_subcores
    bpw = _NROWS // nw
    nch = bpw // _SC_CHUNK
    mesh = plsc.VectorSubcoreMesh(core_axis_name="c", subcore_axis_name="s")

    @functools.partial(
        pl.kernel, mesh=mesh,
        out_type=jax.ShapeDtypeStruct((_NROWS, E_DIM), jnp.float32),
        scratch_types=[
            pltpu.VMEM((_SC_CHUNK,), jnp.int32),
            pltpu.VMEM((_SC_CHUNK, E_DIM), jnp.float32),
            pltpu.SemaphoreType.DMA,
        ],
    )
    def gather_k(table_hbm, idx_hbm, out_hbm, idx_v, rows_v, sem):
        wid = jax.lax.axis_index("s") * info.num_cores + jax.lax.axis_index("c")
        base = wid * bpw
        for c in range(nch):
            off = base + c * _SC_CHUNK
            pltpu.sync_copy(idx_hbm.at[pl.ds(off, _SC_CHUNK)], idx_v)
            pltpu.async_copy(table_hbm.at[idx_v], rows_v, sem).wait()
            pltpu.sync_copy(rows_v, out_hbm.at[pl.ds(off, _SC_CHUNK)])

    return gather_k


def kernel(z, emb_text, emb_graph):
    z_flat = z.reshape(-1, E_DIM)
    zf_text = z_flat[:, :ST]
    zf_graph = z_flat[:, ST:]
    zf_text_n = _l2n(zf_text)
    zf_graph_n = _l2n(zf_graph)
    et_n = _l2n(emb_text)
    eg_n = _l2n(emb_graph)

    stt = jnp.sum(zf_text_n ** 2, axis=1, keepdims=True)
    stg = jnp.sum(zf_graph_n ** 2, axis=1, keepdims=True)
    se_t = jnp.sum(et_n ** 2, axis=1).reshape(1, N_E)
    se_g = jnp.sum(eg_n ** 2, axis=1).reshape(1, N_E)
    sz = stt + stg
    sec = se_t + se_g
    znc = jnp.concatenate([zf_text_n, zf_graph_n], axis=-1)
    all_emb = jnp.concatenate([et_n, eg_n], axis=-1)

    idx3, scal = pl.pallas_call(
        _vq_body,
        grid=(_NB,),
        in_specs=[
            pl.BlockSpec((_R, E_DIM), lambda i: (i, 0)),
            pl.BlockSpec((N_E, E_DIM), lambda i: (0, 0)),
            pl.BlockSpec((_R, 1), lambda i: (i, 0)),
            pl.BlockSpec((_R, 1), lambda i: (i, 0)),
            pl.BlockSpec((_R, 1), lambda i: (i, 0)),
            pl.BlockSpec((1, N_E), lambda i: (0, 0)),
            pl.BlockSpec((1, N_E), lambda i: (0, 0)),
            pl.BlockSpec((1, N_E), lambda i: (0, 0)),
        ],
        out_specs=[
            pl.BlockSpec((1, 1, _R), lambda i: (i, 0, 0)),
            pl.BlockSpec(memory_space=pltpu.SMEM),
        ],
        out_shape=[
            jax.ShapeDtypeStruct((_NB, 1, _R), jnp.int32),
            jax.ShapeDtypeStruct((8,), jnp.float32),
        ],
        scratch_shapes=[
            pltpu.VMEM((1, N_E), jnp.float32),
            pltpu.SMEM((8,), jnp.float32),
            pltpu.VMEM((ST, ST), jnp.float32),
            pltpu.VMEM((ST, ST), jnp.float32),
            pltpu.VMEM((4, ST), jnp.float32),
            pltpu.SMEM((4,), jnp.float32),
            pltpu.VMEM((1, N_E), jnp.float32),
        ],
    )(znc, -2.0 * all_emb, stt, stg, sz, se_t, se_g, sec)

    min_idx = idx3.reshape(_NROWS)
    zq = _make_sc_gather()(all_emb, min_idx)

    z_q_out = zq.reshape(z.shape)
    z_q_text = zq[:, :ST].reshape(z.shape[0], z.shape[1], ST)
    z_q_graph = zq[:, ST:].reshape(z.shape[0], z.shape[1], ST)
    return (z_q_out, scal[0], scal[1], scal[2], scal[3], scal[4],
            z_q_text, z_q_graph, min_idx)


# speedup vs baseline: 1.0951x; 1.0753x over previous
import functools

import jax
import jax.numpy as jnp
from jax.experimental import pallas as pl
from jax.experimental.pallas import tpu as pltpu
from jax.experimental.pallas import tpu_sc as plsc

N_E = 8192
E_DIM = 256
ST = 128
BETA = 0.25
ENT_RATIO = 0.1

_R = 128
_NROWS = 16384
_NB = _NROWS // _R
_SC_CHUNK = 128


def _l2n(x):
    return x / jnp.clip(jnp.linalg.norm(x, axis=-1, keepdims=True), 1e-12, None)


def _vq_body(zraw_ref, ecm2_ref,
             set_ref, seg_ref, sec_ref,
             idx_ref, scal_ref,
             accp_s, sacc_s, gt_s, gg_s, esw_s, ssc_s, iota_s):
    i = pl.program_id(0)

    @pl.when(i == 0)
    def _init():
        accp_s[...] = jnp.zeros_like(accp_s)
        sacc_s[0] = 0.0
        sacc_s[1] = 0.0
        sacc_s[2] = 0.0
        sacc_s[3] = 0.0
        et = ecm2_ref[:, :ST]
        eg = ecm2_ref[:, ST:]
        iota_s[...] = jax.lax.broadcasted_iota(
            jnp.int32, (1, N_E), 1).astype(jnp.float32)
        gt_s[...] = jax.lax.dot_general(et, et, (((0,), (0,)), ((), ())),
                                        preferred_element_type=jnp.float32)
        gg_s[...] = jax.lax.dot_general(eg, eg, (((0,), (0,)), ((), ())),
                                        preferred_element_type=jnp.float32)
        ones_r = jnp.ones((1, N_E), jnp.float32)
        esw_s[0:1, :] = jax.lax.dot_general(
            ones_r, et, (((1,), (0,)), ((), ())),
            preferred_element_type=jnp.float32)
        esw_s[1:2, :] = jax.lax.dot_general(
            set_ref[...], et, (((1,), (0,)), ((), ())),
            preferred_element_type=jnp.float32)
        esw_s[2:3, :] = jax.lax.dot_general(
            ones_r, eg, (((1,), (0,)), ((), ())),
            preferred_element_type=jnp.float32)
        esw_s[3:4, :] = jax.lax.dot_general(
            seg_ref[...], eg, (((1,), (0,)), ((), ())),
            preferred_element_type=jnp.float32)
        se_t = set_ref[...]
        se_g = seg_ref[...]
        ssc_s[0] = jnp.sum(se_t)
        ssc_s[1] = jnp.sum(se_t * se_t)
        ssc_s[2] = jnp.sum(se_g)
        ssc_s[3] = jnp.sum(se_g * se_g)

    zraw = zraw_ref[...]
    zt = zraw[:, :ST]
    zg = zraw[:, ST:]
    nt = jnp.clip(jnp.sqrt(jnp.sum(zt * zt, axis=1, keepdims=True)),
                  1e-12, None)
    ng = jnp.clip(jnp.sqrt(jnp.sum(zg * zg, axis=1, keepdims=True)),
                  1e-12, None)
    ztn = zt / nt
    zgn = zg / ng
    znc = jnp.concatenate([ztn, zgn], axis=1)
    stt_row = jnp.sum(ztn * ztn, axis=1, keepdims=True)
    stg_row = jnp.sum(zgn * zgn, axis=1, keepdims=True)
    sz = stt_row + stg_row
    ip = jax.lax.dot_general(znc, ecm2_ref[...], (((1,), (1,)), ((), ())),
                             preferred_element_type=jnp.float32)
    d = (sz + sec_ref[...]) + ip

    stt_v = stt_row
    stg_v = stg_row
    ut = jax.lax.dot_general(ztn, esw_s[0:2, :], (((1,), (1,)), ((), ())),
                             preferred_element_type=jnp.float32)
    ug = jax.lax.dot_general(zgn, esw_s[2:4, :], (((1,), (1,)), ((), ())),
                             preferred_element_type=jnp.float32)
    ht = jax.lax.dot_general(ztn, gt_s[...], (((1,), (0,)), ((), ())),
                             preferred_element_type=jnp.float32)
    hg = jax.lax.dot_general(zgn, gg_s[...], (((1,), (0,)), ((), ())),
                             preferred_element_type=jnp.float32)
    cross_t = jnp.sum(stt_v * ut[:, 0:1]) + jnp.sum(ut[:, 1:2])
    cross_g = jnp.sum(stg_v * ug[:, 0:1]) + jnp.sum(ug[:, 1:2])
    gram_t = jnp.sum(ht * ztn)
    gram_g = jnp.sum(hg * zgn)
    pdt = (N_E * jnp.sum(stt_v * stt_v) + _R * ssc_s[1]
           + 2.0 * jnp.sum(stt_v) * ssc_s[0]) + 2.0 * cross_t + gram_t
    pdg = (N_E * jnp.sum(stg_v * stg_v) + _R * ssc_s[3]
           + 2.0 * jnp.sum(stg_v) * ssc_s[2]) + 2.0 * cross_g + gram_g

    minv = jnp.min(d, axis=1, keepdims=True)
    cand = jnp.where(d == minv, iota_s[...], float(N_E))
    midx = jnp.min(cand, axis=1).astype(jnp.int32)
    idx_ref[...] = midx.reshape(1, 1, _R)

    pvq = jnp.sum(minv)

    md = minv - d
    e = jnp.exp(md * 100.0)
    s = jnp.sum(e, axis=1, keepdims=True)
    t = jnp.sum(e * md, axis=1, keepdims=True)
    rinv = 1.0 / s
    plogp = t * rinv * 100.0 - jnp.log(s)
    psum = jnp.sum(plogp)
    accp_s[...] += jax.lax.dot_general(rinv, e, (((0,), (0,)), ((), ())),
                                       preferred_element_type=jnp.float32)

    sacc_s[0] += pdt
    sacc_s[1] += pdg
    sacc_s[2] += pvq
    sacc_s[3] += psum

    @pl.when(i == _NB - 1)
    def _fin():
        avg = accp_s[...] * (1.0 / _NROWS)
        avg_ent = -jnp.sum(avg * jnp.log(avg + 1e-05))
        sample_ent = -(sacc_s[3] / _NROWS)
        vq = sacc_s[2] / (_NROWS * 256.0)
        scal_ref[0] = vq
        scal_ref[1] = BETA * vq
        scal_ref[2] = ENT_RATIO * (sample_ent - avg_ent)
        scal_ref[3] = sacc_s[0] / _NROWS
        scal_ref[4] = sacc_s[1] / _NROWS


def _make_sc_gather():
    info = plsc.get_sparse_core_info()
    nw = info.num_cores * info.num_subcores
    bpw = _NROWS // nw
    nch = bpw // _SC_CHUNK
    mesh = plsc.VectorSubcoreMesh(core_axis_name="c", subcore_axis_name="s")

    @functools.partial(
        pl.kernel, mesh=mesh,
        out_type=jax.ShapeDtypeStruct((_NROWS, E_DIM), jnp.float32),
        scratch_types=[
            pltpu.VMEM((_SC_CHUNK,), jnp.int32),
            pltpu.VMEM((_SC_CHUNK, E_DIM), jnp.float32),
            pltpu.SemaphoreType.DMA,
        ],
    )
    def gather_k(table_hbm, idx_hbm, out_hbm, idx_v, rows_v, sem):
        wid = jax.lax.axis_index("s") * info.num_cores + jax.lax.axis_index("c")
        base = wid * bpw
        for c in range(nch):
            off = base + c * _SC_CHUNK
            pltpu.sync_copy(idx_hbm.at[pl.ds(off, _SC_CHUNK)], idx_v)
            pltpu.async_copy(table_hbm.at[idx_v], rows_v, sem).wait()
            pltpu.sync_copy(rows_v, out_hbm.at[pl.ds(off, _SC_CHUNK)])

    return gather_k


def kernel(z, emb_text, emb_graph):
    z_flat = z.reshape(-1, E_DIM)
    et_n = _l2n(emb_text)
    eg_n = _l2n(emb_graph)

    se_t = jnp.sum(et_n ** 2, axis=1).reshape(1, N_E)
    se_g = jnp.sum(eg_n ** 2, axis=1).reshape(1, N_E)
    sec = se_t + se_g
    all_emb = jnp.concatenate([et_n, eg_n], axis=-1)

    idx3, scal = pl.pallas_call(
        _vq_body,
        grid=(_NB,),
        in_specs=[
            pl.BlockSpec((_R, E_DIM), lambda i: (i, 0)),
            pl.BlockSpec((N_E, E_DIM), lambda i: (0, 0)),
            pl.BlockSpec((1, N_E), lambda i: (0, 0)),
            pl.BlockSpec((1, N_E), lambda i: (0, 0)),
            pl.BlockSpec((1, N_E), lambda i: (0, 0)),
        ],
        out_specs=[
            pl.BlockSpec((1, 1, _R), lambda i: (i, 0, 0)),
            pl.BlockSpec(memory_space=pltpu.SMEM),
        ],
        out_shape=[
            jax.ShapeDtypeStruct((_NB, 1, _R), jnp.int32),
            jax.ShapeDtypeStruct((8,), jnp.float32),
        ],
        scratch_shapes=[
            pltpu.VMEM((1, N_E), jnp.float32),
            pltpu.SMEM((8,), jnp.float32),
            pltpu.VMEM((ST, ST), jnp.float32),
            pltpu.VMEM((ST, ST), jnp.float32),
            pltpu.VMEM((4, ST), jnp.float32),
            pltpu.SMEM((4,), jnp.float32),
            pltpu.VMEM((1, N_E), jnp.float32),
        ],
    )(z_flat, -2.0 * all_emb, se_t, se_g, sec)

    min_idx = idx3.reshape(_NROWS)
    zq = _make_sc_gather()(all_emb, min_idx)

    z_q_out = zq.reshape(z.shape)
    z_q_text = zq[:, :ST].reshape(z.shape[0], z.shape[1], ST)
    z_q_graph = zq[:, ST:].reshape(z.shape[0], z.shape[1], ST)
    return (z_q_out, scal[0], scal[1], scal[2], scal[3], scal[4],
            z_q_text, z_q_graph, min_idx)
